# baseline (speedup 1.0000x reference)
import jax
import jax.numpy as jnp
from jax.experimental import pallas as pl
from jax.experimental.pallas import tpu as pltpu

NCHUNK = 6


def _step(x_ref, e_ref, hin_ref, fcW_ref, fcb_ref, W0_ref, b0_ref, W1_ref,
          b1_ref, Wg_ref, bg_ref, Uru_ref, Uc_ref, out_ref, h_s):
    t = pl.program_id(0)
    b = pl.program_id(1)
    F = Uc_ref.shape[0]

    A = e_ref[0, 0]
    x = x_ref[0, 0]

    dot = lambda a, w: jnp.dot(a, w, preferred_element_type=jnp.float32)

    xi = jnp.maximum(dot(x, fcW_ref[...]) + fcb_ref[...], 0.0)
    z = jnp.maximum(dot(dot(A, xi), W0_ref[...]) + b0_ref[...], 0.0)
    z = jnp.maximum(dot(dot(A, z), W1_ref[...]) + b1_ref[...], 0.0)

    h = jnp.where(t == 0, hin_ref[b], h_s[b])
    g = dot(z, Wg_ref[...]) + bg_ref[...]
    g_ru = g[:, : 2 * F] + dot(h, Uru_ref[...])
    ru = jax.nn.sigmoid(g_ru)
    r = ru[:, :F]
    u = ru[:, F:]
    c = jnp.tanh(g[:, 2 * F:] + dot(r * h, Uc_ref[...]))
    hn = u * h + (1.0 - u) * c

    h_s[b] = hn
    out_ref[0, 0] = hn


def kernel(x, e, fc_W, fc_b, W0, b0, W1, b1, Wr, Ur, br, Wu, Uu, bu, Wc, Uc, bc):
    B, T, N, in_ft = x.shape
    out_ft = Ur.shape[0]

    Wg = jnp.concatenate([Wr, Wu, Wc], axis=1)
    bg = jnp.concatenate([br, bu, bc]).reshape(1, -1)
    Uru = jnp.concatenate([Ur, Uu], axis=1)

    row = lambda v: v.reshape(1, -1)
    wspec = lambda s: pl.BlockSpec(s, lambda t, b: (0, 0))
    Tc = T // NCHUNK

    call = pl.pallas_call(
        _step,
        grid=(Tc, B),
        in_specs=[
            pl.BlockSpec((1, 1, N, in_ft), lambda t, b: (b, t, 0, 0)),
            pl.BlockSpec((1, 1, N, N), lambda t, b: (b, t, 0, 0)),
            pl.BlockSpec((B, N, out_ft), lambda t, b: (0, 0, 0)),
            wspec(fc_W.shape), wspec((1, fc_b.shape[0])),
            wspec(W0.shape), wspec((1, b0.shape[0])),
            wspec(W1.shape), wspec((1, b1.shape[0])),
            wspec(Wg.shape), wspec(bg.shape), wspec(Uru.shape), wspec(Uc.shape),
        ],
        out_specs=pl.BlockSpec((1, 1, N, out_ft), lambda t, b: (b, t, 0, 0)),
        out_shape=jax.ShapeDtypeStruct((B, Tc, N, out_ft), jnp.float32),
        scratch_shapes=[pltpu.VMEM((B, N, out_ft), jnp.float32)],
    )

    h = jnp.zeros((B, N, out_ft), jnp.float32)
    outs = []
    for k in range(NCHUNK):
        ek = e[:, k * Tc:(k + 1) * Tc, :, :, 0]
        xk = x[:, k * Tc:(k + 1) * Tc]
        ok = call(xk, ek, h, fc_W, row(fc_b), W0, row(b0), W1, row(b1),
                  Wg, bg, Uru, Uc)
        h = ok[:, -1]
        outs.append(ok)
    return jnp.concatenate(outs, axis=1)


# speedup vs baseline: 1.5574x; 1.5574x over previous
import jax
import jax.numpy as jnp
from jax.experimental import pallas as pl
from jax.experimental.pallas import tpu as pltpu


def _step(x_ref, e_ref, fcW_ref, fcb_ref, W0_ref, b0_ref, W1_ref, b1_ref,
          Wg_ref, bg_ref, Uru_ref, Uc_ref, out_ref, h_s):
    t = pl.program_id(0)
    b = pl.program_id(1)
    F = Uc_ref.shape[0]

    A = e_ref[0, 0]
    x = x_ref[0, 0]

    dot = lambda a, w: jnp.dot(a, w, preferred_element_type=jnp.float32)

    xi = jnp.maximum(dot(x, fcW_ref[...]) + fcb_ref[...], 0.0)
    z = jnp.maximum(dot(dot(A, xi.astype(jnp.bfloat16)), W0_ref[...])
                    + b0_ref[...], 0.0)
    z = jnp.maximum(dot(dot(A, z.astype(jnp.bfloat16)), W1_ref[...])
                    + b1_ref[...], 0.0)

    h = jnp.where(t == 0, 0.0, h_s[b])
    g = dot(z, Wg_ref[...]) + bg_ref[...]
    g_ru = g[:, : 2 * F] + dot(h, Uru_ref[...])
    ru = jax.nn.sigmoid(g_ru)
    r = ru[:, :F]
    u = ru[:, F:]
    c = jnp.tanh(g[:, 2 * F:] + dot(r * h, Uc_ref[...]))
    hn = u * h + (1.0 - u) * c

    h_s[b] = hn
    out_ref[0, 0] = hn


def kernel(x, e, fc_W, fc_b, W0, b0, W1, b1, Wr, Ur, br, Wu, Uu, bu, Wc, Uc, bc):
    B, T, N, in_ft = x.shape
    out_ft = Ur.shape[0]
    A = e[..., 0].astype(jnp.bfloat16)

    Wg = jnp.concatenate([Wr, Wu, Wc], axis=1)
    bg = jnp.concatenate([br, bu, bc]).reshape(1, -1)
    Uru = jnp.concatenate([Ur, Uu], axis=1)

    row = lambda v: v.reshape(1, -1)
    wspec = lambda s: pl.BlockSpec(s, lambda t, b: (0, 0))

    grid = (T, B)
    out = pl.pallas_call(
        _step,
        grid=grid,
        in_specs=[
            pl.BlockSpec((1, 1, N, in_ft), lambda t, b: (b, t, 0, 0)),
            pl.BlockSpec((1, 1, N, N), lambda t, b: (b, t, 0, 0)),
            wspec(fc_W.shape), wspec((1, fc_b.shape[0])),
            wspec(W0.shape), wspec((1, b0.shape[0])),
            wspec(W1.shape), wspec((1, b1.shape[0])),
            wspec(Wg.shape), wspec(bg.shape), wspec(Uru.shape), wspec(Uc.shape),
        ],
        out_specs=pl.BlockSpec((1, 1, N, out_ft), lambda t, b: (b, t, 0, 0)),
        out_shape=jax.ShapeDtypeStruct((B, T, N, out_ft), jnp.float32),
        scratch_shapes=[pltpu.VMEM((B, N, out_ft), jnp.float32)],
    )(x, A, fc_W, row(fc_b), W0, row(b0), W1, row(b1), Wg, bg, Uru, Uc)
    return out


# speedup vs baseline: 2.0132x; 1.2927x over previous
import jax
import jax.numpy as jnp
from jax.experimental import pallas as pl
from jax.experimental.pallas import tpu as pltpu


def _step(xT_ref, e_ref, fcWT_ref, fcb_ref, W0_ref, b0_ref, W1_ref, b1_ref,
          Wg_ref, bg_ref, Uru_ref, Uc_ref, out_ref, h_s):
    t = pl.program_id(0)
    B = xT_ref.shape[0]
    F = Uc_ref.shape[0]

    dot = lambda a, w: jnp.dot(a, w, preferred_element_type=jnp.float32)
    dot_nt = lambda a, bt: jax.lax.dot_general(
        a, bt, (((1,), (1,)), ((), ())), preferred_element_type=jnp.float32)

    for b in range(B):
        A = e_ref[b, 0]
        xT = xT_ref[b, 0]

        xiT = jnp.maximum(dot(fcWT_ref[...], xT) + fcb_ref[...], 0.0)
        z = jnp.maximum(dot(dot_nt(A, xiT.astype(jnp.bfloat16)), W0_ref[...])
                        + b0_ref[...], 0.0)
        z = jnp.maximum(dot(dot(A, z.astype(jnp.bfloat16)), W1_ref[...])
                        + b1_ref[...], 0.0)

        h = jnp.where(t == 0, 0.0, h_s[b])
        g = dot(z, Wg_ref[...]) + bg_ref[...]
        g_ru = g[:, : 2 * F] + dot(h, Uru_ref[...])
        ru = jax.nn.sigmoid(g_ru)
        r = ru[:, :F]
        u = ru[:, F:]
        c = jnp.tanh(g[:, 2 * F:] + dot(r * h, Uc_ref[...]))
        hn = u * h + (1.0 - u) * c

        h_s[b] = hn
        out_ref[b, 0] = hn


def kernel(x, e, fc_W, fc_b, W0, b0, W1, b1, Wr, Ur, br, Wu, Uu, bu, Wc, Uc, bc):
    B, T, N, in_ft = x.shape
    out_ft = Ur.shape[0]
    A = e[..., 0].astype(jnp.bfloat16)
    xT = jnp.transpose(x, (0, 1, 3, 2))

    Wg = jnp.concatenate([Wr, Wu, Wc], axis=1)
    bg = jnp.concatenate([br, bu, bc]).reshape(1, -1)
    Uru = jnp.concatenate([Ur, Uu], axis=1)

    row = lambda v: v.reshape(1, -1)
    wspec = lambda s: pl.BlockSpec(s, lambda t: (0, 0))

    out = pl.pallas_call(
        _step,
        grid=(T,),
        in_specs=[
            pl.BlockSpec((B, 1, in_ft, N), lambda t: (0, t, 0, 0)),
            pl.BlockSpec((B, 1, N, N), lambda t: (0, t, 0, 0)),
            wspec(fc_W.shape), pl.BlockSpec((in_ft, 1), lambda t: (0, 0)),
            wspec(W0.shape), wspec((1, b0.shape[0])),
            wspec(W1.shape), wspec((1, b1.shape[0])),
            wspec(Wg.shape), wspec(bg.shape), wspec(Uru.shape), wspec(Uc.shape),
        ],
        out_specs=pl.BlockSpec((B, 1, N, out_ft), lambda t: (0, t, 0, 0)),
        out_shape=jax.ShapeDtypeStruct((B, T, N, out_ft), jnp.float32),
        scratch_shapes=[pltpu.VMEM((B, N, out_ft), jnp.float32)],
    )(xT, A, fc_W.T, fc_b.reshape(-1, 1), W0, row(b0), W1, row(b1),
      Wg, bg, Uru, Uc)
    return out


# speedup vs baseline: 2.1879x; 1.0868x over previous
import jax
import jax.numpy as jnp
from jax.experimental import pallas as pl
from jax.experimental.pallas import tpu as pltpu


def _step(xT_ref, e_ref, fcWT_ref, fcb_ref, W0_ref, b0_ref, W1_ref, b1_ref,
          Wg_ref, bg_ref, Uru_ref, Uc_ref, out_ref, h_s):
    t = pl.program_id(0)
    B = xT_ref.shape[0]
    F = Uc_ref.shape[0]

    dot = lambda a, w: jnp.dot(a, w, preferred_element_type=jnp.float32)
    dot_nt = lambda a, bt: jax.lax.dot_general(
        a, bt, (((1,), (1,)), ((), ())), preferred_element_type=jnp.float32)

    for b in range(B):
        A = e_ref[b, 0]
        xT = xT_ref[b, 0]

        xiT = jnp.maximum(dot(fcWT_ref[...], xT) + fcb_ref[...], 0.0)
        z = jnp.maximum(dot(dot_nt(A, xiT.astype(jnp.bfloat16)), W0_ref[...])
                        + b0_ref[...], 0.0)
        z = jnp.maximum(dot(dot(A, z.astype(jnp.bfloat16)), W1_ref[...])
                        + b1_ref[...], 0.0)

        h = jnp.where(t == 0, 0.0, h_s[b])
        g = dot(z, Wg_ref[...]) + bg_ref[...]
        g_ru = g[:, : 2 * F] + dot(h, Uru_ref[...])
        ru = jax.nn.sigmoid(g_ru)
        r = ru[:, :F]
        u = ru[:, F:]
        c = jnp.tanh(g[:, 2 * F:] + dot(r * h, Uc_ref[...]))
        hn = u * h + (1.0 - u) * c

        h_s[b] = hn
        out_ref[b, 0] = hn.T


def kernel(x, e, fc_W, fc_b, W0, b0, W1, b1, Wr, Ur, br, Wu, Uu, bu, Wc, Uc, bc):
    B, T, N, in_ft = x.shape
    out_ft = Ur.shape[0]
    A = e[..., 0].astype(jnp.bfloat16)
    xT = jnp.transpose(x, (0, 1, 3, 2))

    Wg = jnp.concatenate([Wr, Wu, Wc], axis=1)
    bg = jnp.concatenate([br, bu, bc]).reshape(1, -1)
    Uru = jnp.concatenate([Ur, Uu], axis=1)

    row = lambda v: v.reshape(1, -1)
    wspec = lambda s: pl.BlockSpec(s, lambda t: (0, 0))

    out = pl.pallas_call(
        _step,
        grid=(T,),
        in_specs=[
            pl.BlockSpec((B, 1, in_ft, N), lambda t: (0, t, 0, 0)),
            pl.BlockSpec((B, 1, N, N), lambda t: (0, t, 0, 0)),
            wspec(fc_W.shape), pl.BlockSpec((in_ft, 1), lambda t: (0, 0)),
            wspec(W0.shape), wspec((1, b0.shape[0])),
            wspec(W1.shape), wspec((1, b1.shape[0])),
            wspec(Wg.shape), wspec(bg.shape), wspec(Uru.shape), wspec(Uc.shape),
        ],
        out_specs=pl.BlockSpec((B, 1, out_ft, N), lambda t: (0, t, 0, 0)),
        out_shape=jax.ShapeDtypeStruct((B, T, out_ft, N), jnp.float32),
        scratch_shapes=[pltpu.VMEM((B, N, out_ft), jnp.float32)],
    )(xT, A, fc_W.T, fc_b.reshape(-1, 1), W0, row(b0), W1, row(b1),
      Wg, bg, Uru, Uc)
    return jnp.transpose(out, (0, 1, 3, 2))
